# baseline (speedup 1.0000x reference)
import functools

import jax
import jax.numpy as jnp
from jax import lax
from jax.experimental import pallas as pl
from jax.experimental.pallas import tpu as pltpu
from jax.experimental.pallas import tpu_sc as plsc

N = 10000
E = 320000
D = 128
B = 256
AV = 128
BV = 16
NA = 9
NB = 3

NC = 2
NS = 16
NW = NC * NS

EG = 128
N_EGROUPS = 2560
EPAD = N_EGROUPS * EG
EPW = N_EGROUPS // NW
N_NGROUPS = -(-N // EG)
NPAD = N_NGROUPS * EG


def _prep_body(at_ref, bt_ref, we_ref, be_ref, wn_ref, bn_ref, gt_ref,
               e0_ref, e1_ref, e2_ref, nt_ref,
               taw_ref, tbc_ref, glat_ref, cidx_ref, nidx_ref):
    wn = wn_ref[...]
    bn = bn_ref[...]
    for f in range(NA):
        r = jnp.dot(at_ref[f], wn, preferred_element_type=jnp.float32)
        if f == 0:
            r = r + bn
        taw_ref[pl.ds(f * AV, AV), :] = r

    we = we_ref[...]
    be = be_ref[...]
    t0 = jnp.dot(bt_ref[0], we, preferred_element_type=jnp.float32) + be
    t1 = jnp.dot(bt_ref[1], we, preferred_element_type=jnp.float32)
    t2 = jnp.dot(bt_ref[2], we, preferred_element_type=jnp.float32)
    for i1 in range(BV):
        t01 = t0 + t1[i1:i1 + 1, :]
        for i2 in range(BV):
            tbc_ref[pl.ds(i2 * 256 + i1 * 16, BV), :] = t01 + t2[i2:i2 + 1, :]

    glat_ref[...] = jnp.broadcast_to(gt_ref[...], (B, D))
    cidx_ref[...] = e0_ref[...] + 16 * e1_ref[...] + 256 * e2_ref[...]
    fld = lax.broadcasted_iota(jnp.int32, (NA * N_NGROUPS, D), 0) // N_NGROUPS
    nidx_ref[...] = nt_ref[...] + AV * fld


NBUF = 1


def _relu_rows(rows_v):
    def relu_row(r, c):
        for j in range(D // 16):
            sl = pl.ds(j * 16, 16)
            rows_v[r, sl] = jnp.maximum(rows_v[r, sl], 0.0)
        return c

    lax.fori_loop(0, EG, relu_row, 0)


def _sc_body(tbc, taw, cidx, nidx, eout, nout,
             i0, i1, i2, i3, r0, r1, r2, r3, s0, s1, s2, s3):
    cid = lax.axis_index("c")
    sid = lax.axis_index("s")
    wid = sid * NC + cid
    idx_b = [i0, i1, i2, i3]
    row_b = [r0, r1, r2, r3]
    sem_b = [s0, s1, s2, s3]

    def edge_quad(i, carry):
        descs = []
        for k in range(NBUF):
            base = (wid + (i * NBUF + k) * NW) * EG
            pltpu.sync_copy(cidx.at[pl.ds(base, EG)], idx_b[k])
            descs.append(pltpu.async_copy(tbc.at[idx_b[k]], row_b[k],
                                          sem_b[k]))
        for k in range(NBUF):
            base = (wid + (i * NBUF + k) * NW) * EG
            descs[k].wait()
            _relu_rows(row_b[k])
            pltpu.sync_copy(row_b[k], eout.at[pl.ds(base, EG)])
        return carry

    lax.fori_loop(0, EPW // NBUF, edge_quad, 0)

    def node_group(i, carry):
        g = wid + i * NW
        pltpu.sync_copy(nidx.at[pl.ds(g * EG, EG)], idx_b[0])
        pltpu.async_copy(taw.at[idx_b[0]], row_b[0], sem_b[0]).wait()
        for f in range(1, NA):
            pltpu.sync_copy(nidx.at[pl.ds((f * N_NGROUPS + g) * EG, EG)],
                            idx_b[1])
            pltpu.async_copy(taw.at[idx_b[1]], row_b[1], sem_b[1]).wait()

            def add_row(r, c):
                for j in range(D // 16):
                    sl = pl.ds(j * 16, 16)
                    row_b[0][r, sl] = row_b[0][r, sl] + row_b[1][r, sl]
                return c

            lax.fori_loop(0, EG, add_row, 0)
        _relu_rows(row_b[0])
        pltpu.sync_copy(row_b[0], nout.at[pl.ds(g * EG, EG)])
        return carry

    my_ng = (N_NGROUPS - wid + NW - 1) // NW
    lax.fori_loop(0, my_ng, node_group, 0)


def kernel(nodes, edges, receivers, senders, node_graph_idx, edge_graph_idx,
           atom_tables, bond_tables, W_edge, b_edge, W_node, b_node,
           global_table):
    epad = jnp.pad(edges, ((0, EPAD - E), (0, 0)))
    e0 = epad[:, 0].reshape(EPAD // D, D)
    e1 = epad[:, 1].reshape(EPAD // D, D)
    e2 = epad[:, 2].reshape(EPAD // D, D)
    nodes_t = jnp.pad(nodes.T, ((0, 0), (0, NPAD - N))).reshape(
        NA * N_NGROUPS, EG)

    taw, tbc, glat, cidx2d, nidx2d = pl.pallas_call(
        _prep_body,
        out_shape=(
            jax.ShapeDtypeStruct((NA * AV, D), jnp.float32),
            jax.ShapeDtypeStruct((BV * BV * BV, D), jnp.float32),
            jax.ShapeDtypeStruct((B, D), jnp.float32),
            jax.ShapeDtypeStruct((EPAD // D, D), jnp.int32),
            jax.ShapeDtypeStruct((NA * N_NGROUPS, EG), jnp.int32),
        ),
    )(atom_tables, bond_tables, W_edge, b_edge.reshape(1, D),
      W_node, b_node.reshape(1, D), global_table,
      e0, e1, e2, nodes_t)

    cidx = cidx2d.reshape(EPAD)
    nidx = nidx2d.reshape(NA * NPAD)

    mesh = plsc.VectorSubcoreMesh(core_axis_name="c", subcore_axis_name="s",
                                  num_cores=NC, num_subcores=NS)
    sc = functools.partial(
        pl.kernel,
        out_type=(
            jax.ShapeDtypeStruct((EPAD, D), jnp.float32),
            jax.ShapeDtypeStruct((NPAD, D), jnp.float32),
        ),
        mesh=mesh,
        scratch_types=(
            [pltpu.VMEM((EG,), jnp.int32)] * 4
            + [pltpu.VMEM((EG, D), jnp.float32)] * 4
            + [pltpu.SemaphoreType.DMA] * 4
        ),
    )(_sc_body)

    edges_pad, nodes_pad = sc(tbc, taw, cidx, nidx)
    edges_update = edges_pad[:E]
    nodes_update = nodes_pad[:N]

    return (nodes_update, edges_update, receivers, senders, glat,
            node_graph_idx, edge_graph_idx)


# speedup vs baseline: 1.8918x; 1.8918x over previous
import functools

import jax
import jax.numpy as jnp
from jax import lax
from jax.experimental import pallas as pl
from jax.experimental.pallas import tpu as pltpu
from jax.experimental.pallas import tpu_sc as plsc

N = 10000
E = 320000
D = 128
B = 256
AV = 128
BV = 16
NA = 9
NB = 3

NC = 2
NS = 16
NW = NC * NS

EG = 128
N_EGROUPS = E // EG
N_NGROUPS = -(-N // EG)
NPAD = N_NGROUPS * EG


def _prep_body(at_ref, bt_ref, we_ref, be_ref, wn_ref, bn_ref, gt_ref,
               e0_ref, e1_ref, e2_ref, nt_ref,
               taw_ref, tbc_ref, glat_ref, cidx_ref, nidx_ref):
    wn = wn_ref[...]
    bn = bn_ref[...]
    for f in range(NA):
        r = jnp.dot(at_ref[f], wn, preferred_element_type=jnp.float32)
        if f == 0:
            r = r + bn
        taw_ref[pl.ds(f * AV, AV), :] = r

    we = we_ref[...]
    be = be_ref[...]
    t0 = jnp.dot(bt_ref[0], we, preferred_element_type=jnp.float32) + be
    t1 = jnp.dot(bt_ref[1], we, preferred_element_type=jnp.float32)
    t2 = jnp.dot(bt_ref[2], we, preferred_element_type=jnp.float32)
    for i1 in range(BV):
        t01 = t0 + t1[i1:i1 + 1, :]
        for i2 in range(BV):
            tbc_ref[pl.ds(i2 * 256 + i1 * 16, BV), :] = t01 + t2[i2:i2 + 1, :]

    glat_ref[...] = jnp.broadcast_to(gt_ref[...], (B, D))
    cidx_ref[...] = e0_ref[...] + 16 * e1_ref[...] + 256 * e2_ref[...]
    fld = lax.broadcasted_iota(jnp.int32, (NA * N_NGROUPS, D), 0) // N_NGROUPS
    nidx_ref[...] = nt_ref[...] + AV * fld


def _relu_rows(rows_v):
    def relu_row(r, c):
        for j in range(D // 16):
            sl = pl.ds(j * 16, 16)
            rows_v[r, sl] = jnp.maximum(rows_v[r, sl], 0.0)
        return c

    lax.fori_loop(0, EG, relu_row, 0)


def _sc_body(tbc, taw, cidx, nidx, eout, nout,
             eidx_v, erows_v, nidx_v, nacc_v, ngat_v, sem):
    cid = lax.axis_index("c")
    sid = lax.axis_index("s")
    wid = sid * NC + cid

    def edge_group(i, carry):
        base = (wid + i * NW) * EG
        pltpu.sync_copy(cidx.at[pl.ds(base, EG)], eidx_v)
        pltpu.async_copy(tbc.at[eidx_v], erows_v, sem).wait()
        _relu_rows(erows_v)
        pltpu.sync_copy(erows_v, eout.at[pl.ds(base, EG)])
        return carry

    my_eg = (N_EGROUPS - wid + NW - 1) // NW
    lax.fori_loop(0, my_eg, edge_group, 0)

    def node_group(i, carry):
        g = wid + i * NW
        pltpu.sync_copy(nidx.at[pl.ds(g * EG, EG)], nidx_v)
        pltpu.async_copy(taw.at[nidx_v], nacc_v, sem).wait()
        for f in range(1, NA):
            pltpu.sync_copy(nidx.at[pl.ds((f * N_NGROUPS + g) * EG, EG)],
                            nidx_v)
            pltpu.async_copy(taw.at[nidx_v], ngat_v, sem).wait()

            def add_row(r, c):
                for j in range(D // 16):
                    sl = pl.ds(j * 16, 16)
                    nacc_v[r, sl] = nacc_v[r, sl] + ngat_v[r, sl]
                return c

            lax.fori_loop(0, EG, add_row, 0)
        _relu_rows(nacc_v)
        pltpu.sync_copy(nacc_v, nout.at[pl.ds(g * EG, EG)])
        return carry

    my_ng = (N_NGROUPS - wid + NW - 1) // NW
    lax.fori_loop(0, my_ng, node_group, 0)


def kernel(nodes, edges, receivers, senders, node_graph_idx, edge_graph_idx,
           atom_tables, bond_tables, W_edge, b_edge, W_node, b_node,
           global_table):
    e0 = edges[:, 0].reshape(E // D, D)
    e1 = edges[:, 1].reshape(E // D, D)
    e2 = edges[:, 2].reshape(E // D, D)
    nodes_t = jnp.pad(nodes.T, ((0, 0), (0, NPAD - N))).reshape(
        NA * N_NGROUPS, EG)

    taw, tbc, glat, cidx2d, nidx2d = pl.pallas_call(
        _prep_body,
        out_shape=(
            jax.ShapeDtypeStruct((NA * AV, D), jnp.float32),
            jax.ShapeDtypeStruct((BV * BV * BV, D), jnp.float32),
            jax.ShapeDtypeStruct((B, D), jnp.float32),
            jax.ShapeDtypeStruct((E // D, D), jnp.int32),
            jax.ShapeDtypeStruct((NA * N_NGROUPS, EG), jnp.int32),
        ),
    )(atom_tables, bond_tables, W_edge, b_edge.reshape(1, D),
      W_node, b_node.reshape(1, D), global_table,
      e0, e1, e2, nodes_t)

    cidx = cidx2d.reshape(E)
    nidx = nidx2d.reshape(NA * NPAD)

    mesh = plsc.VectorSubcoreMesh(core_axis_name="c", subcore_axis_name="s",
                                  num_cores=NC, num_subcores=NS)
    sc = functools.partial(
        pl.kernel,
        out_type=(
            jax.ShapeDtypeStruct((E, D), jnp.float32),
            jax.ShapeDtypeStruct((NPAD, D), jnp.float32),
        ),
        mesh=mesh,
        scratch_types=[
            pltpu.VMEM((EG,), jnp.int32),
            pltpu.VMEM((EG, D), jnp.float32),
            pltpu.VMEM((EG,), jnp.int32),
            pltpu.VMEM((EG, D), jnp.float32),
            pltpu.VMEM((EG, D), jnp.float32),
            pltpu.SemaphoreType.DMA,
        ],
    )(_sc_body)

    edges_update, nodes_pad = sc(tbc, taw, cidx, nidx)
    nodes_update = nodes_pad[:N]

    return (nodes_update, edges_update, receivers, senders, glat,
            node_graph_idx, edge_graph_idx)


# speedup vs baseline: 2.2328x; 1.1803x over previous
import functools

import jax
import jax.numpy as jnp
from jax import lax
from jax.experimental import pallas as pl
from jax.experimental.pallas import tpu as pltpu
from jax.experimental.pallas import tpu_sc as plsc

N = 10000
E = 320000
D = 128
B = 256
AV = 128
BV = 16
NA = 9
NB = 3

NC = 2
NS = 16
NW = NC * NS

EG = 128
N_EGROUPS = E // EG
N_NGROUPS = -(-N // EG)
NPAD = N_NGROUPS * EG


def _prep_body(at_ref, bt_ref, we_ref, be_ref, wn_ref, bn_ref, gt_ref,
               e0_ref, e1_ref, e2_ref, nt_ref,
               taw_ref, tbc_ref, glat_ref, cidx_ref, nidx_ref):
    wn = wn_ref[...]
    bn = bn_ref[...]
    for f in range(NA):
        r = jnp.dot(at_ref[f], wn, preferred_element_type=jnp.float32)
        if f == 0:
            r = r + bn
        taw_ref[pl.ds(f * AV, AV), :] = r

    we = we_ref[...]
    be = be_ref[...]
    t0 = jnp.dot(bt_ref[0], we, preferred_element_type=jnp.float32) + be
    t1 = jnp.dot(bt_ref[1], we, preferred_element_type=jnp.float32)
    t2 = jnp.dot(bt_ref[2], we, preferred_element_type=jnp.float32)
    for i1 in range(BV):
        t01 = t0 + t1[i1:i1 + 1, :]
        for i2 in range(BV):
            tbc_ref[pl.ds(i2 * 256 + i1 * 16, BV), :] = t01 + t2[i2:i2 + 1, :]

    glat_ref[...] = jnp.broadcast_to(gt_ref[...], (B, D))
    cidx_ref[...] = e0_ref[...] + 16 * e1_ref[...] + 256 * e2_ref[...]
    fld = lax.broadcasted_iota(jnp.int32, (NA * N_NGROUPS, D), 0) // N_NGROUPS
    nidx_ref[...] = nt_ref[...] + AV * fld


def _relu_rows(rows_v):
    def relu_row(r, c):
        for j in range(D // 16):
            sl = pl.ds(j * 16, 16)
            rows_v[r, sl] = jnp.maximum(rows_v[r, sl], 0.0)
        return c

    lax.fori_loop(0, EG, relu_row, 0)


def _sc_body(tbc, taw, cidx, nidx, eout, nout,
             eidx_v, erows_v, nidx_v, nacc_v, ngat_v, sem, sem2):
    cid = lax.axis_index("c")
    sid = lax.axis_index("s")
    wid = sid * NC + cid

    my_eg = (N_EGROUPS - wid + NW - 1) // NW

    def edge_pair(i, carry):
        b0 = (wid + (2 * i) * NW) * EG
        b1 = (wid + (2 * i + 1) * NW) * EG
        pltpu.sync_copy(cidx.at[pl.ds(b0, EG)], eidx_v)
        d0 = pltpu.async_copy(tbc.at[eidx_v], erows_v, sem)
        pltpu.sync_copy(cidx.at[pl.ds(b1, EG)], nidx_v)
        d1 = pltpu.async_copy(tbc.at[nidx_v], nacc_v, sem2)
        d0.wait()
        _relu_rows(erows_v)
        pltpu.sync_copy(erows_v, eout.at[pl.ds(b0, EG)])
        d1.wait()
        _relu_rows(nacc_v)
        pltpu.sync_copy(nacc_v, eout.at[pl.ds(b1, EG)])
        return carry

    lax.fori_loop(0, my_eg // 2, edge_pair, 0)

    @pl.when(my_eg % 2 == 1)
    def _odd_tail():
        base = (wid + (my_eg - 1) * NW) * EG
        pltpu.sync_copy(cidx.at[pl.ds(base, EG)], eidx_v)
        pltpu.async_copy(tbc.at[eidx_v], erows_v, sem).wait()
        _relu_rows(erows_v)
        pltpu.sync_copy(erows_v, eout.at[pl.ds(base, EG)])

    def node_group(i, carry):
        g = wid + i * NW
        pltpu.sync_copy(nidx.at[pl.ds(g * EG, EG)], nidx_v)
        pltpu.async_copy(taw.at[nidx_v], nacc_v, sem).wait()
        for f in range(1, NA):
            pltpu.sync_copy(nidx.at[pl.ds((f * N_NGROUPS + g) * EG, EG)],
                            nidx_v)
            pltpu.async_copy(taw.at[nidx_v], ngat_v, sem).wait()

            def add_row(r, c):
                for j in range(D // 16):
                    sl = pl.ds(j * 16, 16)
                    nacc_v[r, sl] = nacc_v[r, sl] + ngat_v[r, sl]
                return c

            lax.fori_loop(0, EG, add_row, 0)
        _relu_rows(nacc_v)
        pltpu.sync_copy(nacc_v, nout.at[pl.ds(g * EG, EG)])
        return carry

    my_ng = (N_NGROUPS - wid + NW - 1) // NW
    lax.fori_loop(0, my_ng, node_group, 0)


def kernel(nodes, edges, receivers, senders, node_graph_idx, edge_graph_idx,
           atom_tables, bond_tables, W_edge, b_edge, W_node, b_node,
           global_table):
    e0 = edges[:, 0].reshape(E // D, D)
    e1 = edges[:, 1].reshape(E // D, D)
    e2 = edges[:, 2].reshape(E // D, D)
    nodes_t = jnp.pad(nodes.T, ((0, 0), (0, NPAD - N))).reshape(
        NA * N_NGROUPS, EG)

    taw, tbc, glat, cidx2d, nidx2d = pl.pallas_call(
        _prep_body,
        out_shape=(
            jax.ShapeDtypeStruct((NA * AV, D), jnp.float32),
            jax.ShapeDtypeStruct((BV * BV * BV, D), jnp.float32),
            jax.ShapeDtypeStruct((B, D), jnp.float32),
            jax.ShapeDtypeStruct((E // D, D), jnp.int32),
            jax.ShapeDtypeStruct((NA * N_NGROUPS, EG), jnp.int32),
        ),
    )(atom_tables, bond_tables, W_edge, b_edge.reshape(1, D),
      W_node, b_node.reshape(1, D), global_table,
      e0, e1, e2, nodes_t)

    cidx = cidx2d.reshape(E)
    nidx = nidx2d.reshape(NA * NPAD)

    mesh = plsc.VectorSubcoreMesh(core_axis_name="c", subcore_axis_name="s",
                                  num_cores=NC, num_subcores=NS)
    sc = functools.partial(
        pl.kernel,
        out_type=(
            jax.ShapeDtypeStruct((E, D), jnp.float32),
            jax.ShapeDtypeStruct((NPAD, D), jnp.float32),
        ),
        mesh=mesh,
        scratch_types=[
            pltpu.VMEM((EG,), jnp.int32),
            pltpu.VMEM((EG, D), jnp.float32),
            pltpu.VMEM((EG,), jnp.int32),
            pltpu.VMEM((EG, D), jnp.float32),
            pltpu.VMEM((EG, D), jnp.float32),
            pltpu.SemaphoreType.DMA,
            pltpu.SemaphoreType.DMA,
        ],
    )(_sc_body)

    edges_update, nodes_pad = sc(tbc, taw, cidx, nidx)
    nodes_update = nodes_pad[:N]

    return (nodes_update, edges_update, receivers, senders, glat,
            node_graph_idx, edge_graph_idx)


# speedup vs baseline: 3.0654x; 1.3729x over previous
import functools

import jax
import jax.numpy as jnp
from jax import lax
from jax.experimental import pallas as pl
from jax.experimental.pallas import tpu as pltpu
from jax.experimental.pallas import tpu_sc as plsc

N = 10000
E = 320000
D = 128
B = 256
AV = 128
BV = 16
NA = 9
NB = 3

NC = 2
NS = 16
NW = NC * NS

EG = 128
N_EGROUPS = E // EG
N_NGROUPS = -(-N // EG)
NPAD = N_NGROUPS * EG


def _prep_body(at_ref, bt_ref, we_ref, be_ref, wn_ref, bn_ref, gt_ref,
               e0_ref, e1_ref, e2_ref, nt_ref,
               taw_ref, tbc_ref, glat_ref, cidx_ref, nidx_ref):
    wn = wn_ref[...]
    bn = bn_ref[...]
    for f in range(NA):
        r = jnp.dot(at_ref[f], wn, preferred_element_type=jnp.float32)
        if f == 0:
            r = r + bn
        taw_ref[pl.ds(f * AV, AV), :] = r

    we = we_ref[...]
    be = be_ref[...]
    t0 = jnp.dot(bt_ref[0], we, preferred_element_type=jnp.float32) + be
    t1 = jnp.dot(bt_ref[1], we, preferred_element_type=jnp.float32)
    t2 = jnp.dot(bt_ref[2], we, preferred_element_type=jnp.float32)
    for i1 in range(BV):
        t01 = t0 + t1[i1:i1 + 1, :]
        for i2 in range(BV):
            tbc_ref[pl.ds(i2 * 256 + i1 * 16, BV), :] = t01 + t2[i2:i2 + 1, :]

    glat_ref[...] = jnp.broadcast_to(gt_ref[...], (B, D))
    cidx_ref[...] = e0_ref[...] + 16 * e1_ref[...] + 256 * e2_ref[...]
    fld = lax.broadcasted_iota(jnp.int32, (NA * N_NGROUPS, D), 0) // N_NGROUPS
    nidx_ref[...] = nt_ref[...] + AV * fld


def _relu_rows(rows_v):
    def relu_row(r, c):
        for j in range(D // 16):
            sl = pl.ds(j * 16, 16)
            rows_v[r, sl] = jnp.maximum(rows_v[r, sl], 0.0)
        return c

    lax.fori_loop(0, EG, relu_row, 0)


NEB = 3


def _sc_body(tbc, taw, cidx, nidx, eout, nout,
             ei0, ei1, ei2, er0, er1, er2, ni_v, nacc_v, ngat_v,
             g0, g1, g2, w0, w1, w2, ns0, ns1):
    cid = lax.axis_index("c")
    sid = lax.axis_index("s")
    wid = sid * NC + cid
    idx_b = [ei0, ei1, ei2]
    row_b = [er0, er1, er2]
    gsem = [g0, g1, g2]
    wsem = [w0, w1, w2]

    my_eg = (N_EGROUPS - wid + NW - 1) // NW

    def edge_blk(i, carry):
        descs = []
        for k in range(NEB):
            @pl.when(i > 0)
            def _drain(_k=k):
                pltpu.make_async_copy(row_b[_k], eout.at[pl.ds(0, EG)],
                                      wsem[_k]).wait()

            base = (wid + (i * NEB + k) * NW) * EG
            pltpu.sync_copy(cidx.at[pl.ds(base, EG)], idx_b[k])
            descs.append(pltpu.async_copy(tbc.at[idx_b[k]], row_b[k],
                                          gsem[k]))
        for k in range(NEB):
            base = (wid + (i * NEB + k) * NW) * EG
            descs[k].wait()
            _relu_rows(row_b[k])
            pltpu.async_copy(row_b[k], eout.at[pl.ds(base, EG)], wsem[k])
        return carry

    nblk = my_eg // NEB
    lax.fori_loop(0, nblk, edge_blk, 0)
    for k in range(NEB):
        @pl.when(nblk > 0)
        def _drain_tail(_k=k):
            pltpu.make_async_copy(row_b[_k], eout.at[pl.ds(0, EG)],
                                  wsem[_k]).wait()

    def edge_one(base):
        pltpu.sync_copy(cidx.at[pl.ds(base, EG)], idx_b[0])
        pltpu.async_copy(tbc.at[idx_b[0]], row_b[0], gsem[0]).wait()
        _relu_rows(row_b[0])
        pltpu.sync_copy(row_b[0], eout.at[pl.ds(base, EG)])

    rem = my_eg - nblk * NEB
    for t in range(1, NEB):
        @pl.when(rem >= t)
        def _tail(_t=t):
            edge_one((wid + (nblk * NEB + _t - 1) * NW) * EG)

    def node_group(i, carry):
        g = wid + i * NW
        pltpu.sync_copy(nidx.at[pl.ds(g * EG, EG)], ni_v)
        d0 = pltpu.async_copy(taw.at[ni_v], nacc_v, ns0)
        pltpu.sync_copy(nidx.at[pl.ds((N_NGROUPS + g) * EG, EG)], idx_b[0])
        dprev = pltpu.async_copy(taw.at[idx_b[0]], ngat_v, ns1)
        d0.wait()
        buf = [ngat_v, row_b[0]]
        ibuf = [idx_b[0], idx_b[1]]
        bsem = [ns1, gsem[0]]
        for f in range(2, NA + 1):
            pb = (f - 2) % 2
            if f < NA:
                cb = (f - 1) % 2
                pltpu.sync_copy(nidx.at[pl.ds((f * N_NGROUPS + g) * EG, EG)],
                                ibuf[cb])
                dcur = pltpu.async_copy(taw.at[ibuf[cb]], buf[cb], bsem[cb])
            dprev.wait()

            def add_row(r, c, _b=pb):
                for j in range(D // 16):
                    sl = pl.ds(j * 16, 16)
                    nacc_v[r, sl] = nacc_v[r, sl] + buf[_b][r, sl]
                return c

            lax.fori_loop(0, EG, add_row, 0)
            if f < NA:
                dprev = dcur
        _relu_rows(nacc_v)
        pltpu.sync_copy(nacc_v, nout.at[pl.ds(g * EG, EG)])
        return carry

    my_ng = (N_NGROUPS - wid + NW - 1) // NW
    lax.fori_loop(0, my_ng, node_group, 0)


def kernel(nodes, edges, receivers, senders, node_graph_idx, edge_graph_idx,
           atom_tables, bond_tables, W_edge, b_edge, W_node, b_node,
           global_table):
    e0 = edges[:, 0].reshape(E // D, D)
    e1 = edges[:, 1].reshape(E // D, D)
    e2 = edges[:, 2].reshape(E // D, D)
    nodes_t = jnp.pad(nodes.T, ((0, 0), (0, NPAD - N))).reshape(
        NA * N_NGROUPS, EG)

    taw, tbc, glat, cidx2d, nidx2d = pl.pallas_call(
        _prep_body,
        out_shape=(
            jax.ShapeDtypeStruct((NA * AV, D), jnp.float32),
            jax.ShapeDtypeStruct((BV * BV * BV, D), jnp.float32),
            jax.ShapeDtypeStruct((B, D), jnp.float32),
            jax.ShapeDtypeStruct((E // D, D), jnp.int32),
            jax.ShapeDtypeStruct((NA * N_NGROUPS, EG), jnp.int32),
        ),
    )(atom_tables, bond_tables, W_edge, b_edge.reshape(1, D),
      W_node, b_node.reshape(1, D), global_table,
      e0, e1, e2, nodes_t)

    cidx = cidx2d.reshape(E)
    nidx = nidx2d.reshape(NA * NPAD)

    mesh = plsc.VectorSubcoreMesh(core_axis_name="c", subcore_axis_name="s",
                                  num_cores=NC, num_subcores=NS)
    sc = functools.partial(
        pl.kernel,
        out_type=(
            jax.ShapeDtypeStruct((E, D), jnp.float32),
            jax.ShapeDtypeStruct((NPAD, D), jnp.float32),
        ),
        mesh=mesh,
        scratch_types=(
            [pltpu.VMEM((EG,), jnp.int32)] * 3
            + [pltpu.VMEM((EG, D), jnp.float32)] * 3
            + [pltpu.VMEM((EG,), jnp.int32)]
            + [pltpu.VMEM((EG, D), jnp.float32)] * 2
            + [pltpu.SemaphoreType.DMA] * 8
        ),
    )(_sc_body)

    edges_update, nodes_pad = sc(tbc, taw, cidx, nidx)
    nodes_update = nodes_pad[:N]

    return (nodes_update, edges_update, receivers, senders, glat,
            node_graph_idx, edge_graph_idx)


# speedup vs baseline: 4.0037x; 1.3061x over previous
import functools

import jax
import jax.numpy as jnp
from jax import lax
from jax.experimental import pallas as pl
from jax.experimental.pallas import tpu as pltpu
from jax.experimental.pallas import tpu_sc as plsc

N = 10000
E = 320000
D = 128
B = 256
AV = 128
BV = 16
NA = 9
NB = 3

NC = 2
NS = 16
NW = NC * NS

EG = 128
N_EGROUPS = E // EG
N_NGROUPS = -(-N // EG)
NPAD = N_NGROUPS * EG


def _prep_body(at_ref, bt_ref, we_ref, be_ref, wn_ref, bn_ref, gt_ref,
               e0_ref, e1_ref, e2_ref, nt_ref,
               taw_ref, tbc_ref, glat_ref, cidx_ref, nidx_ref):
    wn = wn_ref[...]
    bn = bn_ref[...]
    for f in range(NA):
        r = jnp.dot(at_ref[f], wn, preferred_element_type=jnp.float32)
        if f == 0:
            r = r + bn
        taw_ref[pl.ds(f * AV, AV), :] = r

    we = we_ref[...]
    be = be_ref[...]
    t0 = jnp.dot(bt_ref[0], we, preferred_element_type=jnp.float32) + be
    t1 = jnp.dot(bt_ref[1], we, preferred_element_type=jnp.float32)
    t2 = jnp.dot(bt_ref[2], we, preferred_element_type=jnp.float32)
    for i1 in range(BV):
        t01 = t0 + t1[i1:i1 + 1, :]
        for i2 in range(BV):
            tbc_ref[pl.ds(i2 * 256 + i1 * 16, BV), :] = t01 + t2[i2:i2 + 1, :]

    glat_ref[...] = jnp.broadcast_to(gt_ref[...], (B, D))
    cidx_ref[...] = e0_ref[...] + 16 * e1_ref[...] + 256 * e2_ref[...]
    fld = lax.broadcasted_iota(jnp.int32, (NA * N_NGROUPS, D), 0) // N_NGROUPS
    nidx_ref[...] = nt_ref[...] + AV * fld


def _relu_rows(rows_v):
    def relu_row(r, c):
        for j in range(D // 16):
            sl = pl.ds(j * 16, 16)
            rows_v[r, sl] = jnp.maximum(rows_v[r, sl], 0.0)
        return c

    lax.fori_loop(0, EG, relu_row, 0)


NEB = 3


def _sc_body(tbc, taw, cidx, nidx, eout, nout,
             ei0, ei1, ei2, er0, er1, er2, ni_v, nacc_v, ngat_v,
             tbc_sh, taw_sh,
             g0, g1, g2, w0, w1, w2, ns0, ns1):
    cid = lax.axis_index("c")
    sid = lax.axis_index("s")
    wid = sid * NC + cid
    idx_b = [ei0, ei1, ei2]
    row_b = [er0, er1, er2]
    gsem = [g0, g1, g2]
    wsem = [w0, w1, w2]

    @pl.when(sid == 0)
    def _stage():
        pltpu.sync_copy(tbc, tbc_sh)
        pltpu.sync_copy(taw, taw_sh)

    plsc.subcore_barrier()

    my_eg = (N_EGROUPS - wid + NW - 1) // NW

    def edge_blk(i, carry):
        descs = []
        for k in range(NEB):
            @pl.when(i > 0)
            def _drain(_k=k):
                pltpu.make_async_copy(row_b[_k], eout.at[pl.ds(0, EG)],
                                      wsem[_k]).wait()

            base = (wid + (i * NEB + k) * NW) * EG
            pltpu.sync_copy(cidx.at[pl.ds(base, EG)], idx_b[k])
            descs.append(pltpu.async_copy(tbc_sh.at[idx_b[k]], row_b[k],
                                          gsem[k]))
        for k in range(NEB):
            base = (wid + (i * NEB + k) * NW) * EG
            descs[k].wait()
            _relu_rows(row_b[k])
            pltpu.async_copy(row_b[k], eout.at[pl.ds(base, EG)], wsem[k])
        return carry

    nblk = my_eg // NEB
    lax.fori_loop(0, nblk, edge_blk, 0)
    for k in range(NEB):
        @pl.when(nblk > 0)
        def _drain_tail(_k=k):
            pltpu.make_async_copy(row_b[_k], eout.at[pl.ds(0, EG)],
                                  wsem[_k]).wait()

    def edge_one(base):
        pltpu.sync_copy(cidx.at[pl.ds(base, EG)], idx_b[0])
        pltpu.async_copy(tbc_sh.at[idx_b[0]], row_b[0], gsem[0]).wait()
        _relu_rows(row_b[0])
        pltpu.sync_copy(row_b[0], eout.at[pl.ds(base, EG)])

    rem = my_eg - nblk * NEB
    for t in range(1, NEB):
        @pl.when(rem >= t)
        def _tail(_t=t):
            edge_one((wid + (nblk * NEB + _t - 1) * NW) * EG)

    def node_group(i, carry):
        g = wid + i * NW
        pltpu.sync_copy(nidx.at[pl.ds(g * EG, EG)], ni_v)
        d0 = pltpu.async_copy(taw_sh.at[ni_v], nacc_v, ns0)
        pltpu.sync_copy(nidx.at[pl.ds((N_NGROUPS + g) * EG, EG)], idx_b[0])
        dprev = pltpu.async_copy(taw_sh.at[idx_b[0]], ngat_v, ns1)
        d0.wait()
        buf = [ngat_v, row_b[0]]
        ibuf = [idx_b[0], idx_b[1]]
        bsem = [ns1, gsem[0]]
        for f in range(2, NA + 1):
            pb = (f - 2) % 2
            if f < NA:
                cb = (f - 1) % 2
                pltpu.sync_copy(nidx.at[pl.ds((f * N_NGROUPS + g) * EG, EG)],
                                ibuf[cb])
                dcur = pltpu.async_copy(taw_sh.at[ibuf[cb]], buf[cb], bsem[cb])
            dprev.wait()

            def add_row(r, c, _b=pb):
                for j in range(D // 16):
                    sl = pl.ds(j * 16, 16)
                    nacc_v[r, sl] = nacc_v[r, sl] + buf[_b][r, sl]
                return c

            lax.fori_loop(0, EG, add_row, 0)
            if f < NA:
                dprev = dcur
        _relu_rows(nacc_v)
        pltpu.sync_copy(nacc_v, nout.at[pl.ds(g * EG, EG)])
        return carry

    my_ng = (N_NGROUPS - wid + NW - 1) // NW
    lax.fori_loop(0, my_ng, node_group, 0)


def kernel(nodes, edges, receivers, senders, node_graph_idx, edge_graph_idx,
           atom_tables, bond_tables, W_edge, b_edge, W_node, b_node,
           global_table):
    e0 = edges[:, 0].reshape(E // D, D)
    e1 = edges[:, 1].reshape(E // D, D)
    e2 = edges[:, 2].reshape(E // D, D)
    nodes_t = jnp.pad(nodes.T, ((0, 0), (0, NPAD - N))).reshape(
        NA * N_NGROUPS, EG)

    taw, tbc, glat, cidx2d, nidx2d = pl.pallas_call(
        _prep_body,
        out_shape=(
            jax.ShapeDtypeStruct((NA * AV, D), jnp.float32),
            jax.ShapeDtypeStruct((BV * BV * BV, D), jnp.float32),
            jax.ShapeDtypeStruct((B, D), jnp.float32),
            jax.ShapeDtypeStruct((E // D, D), jnp.int32),
            jax.ShapeDtypeStruct((NA * N_NGROUPS, EG), jnp.int32),
        ),
    )(atom_tables, bond_tables, W_edge, b_edge.reshape(1, D),
      W_node, b_node.reshape(1, D), global_table,
      e0, e1, e2, nodes_t)

    cidx = cidx2d.reshape(E)
    nidx = nidx2d.reshape(NA * NPAD)

    mesh = plsc.VectorSubcoreMesh(core_axis_name="c", subcore_axis_name="s",
                                  num_cores=NC, num_subcores=NS)
    sc = functools.partial(
        pl.kernel,
        out_type=(
            jax.ShapeDtypeStruct((E, D), jnp.float32),
            jax.ShapeDtypeStruct((NPAD, D), jnp.float32),
        ),
        mesh=mesh,
        scratch_types=(
            [pltpu.VMEM((EG,), jnp.int32)] * 3
            + [pltpu.VMEM((EG, D), jnp.float32)] * 3
            + [pltpu.VMEM((EG,), jnp.int32)]
            + [pltpu.VMEM((EG, D), jnp.float32)] * 2
            + [pltpu.VMEM_SHARED((BV * BV * BV, D), jnp.float32)]
            + [pltpu.VMEM_SHARED((NA * AV, D), jnp.float32)]
            + [pltpu.SemaphoreType.DMA] * 8
        ),
    )(_sc_body)

    edges_update, nodes_pad = sc(tbc, taw, cidx, nidx)
    nodes_update = nodes_pad[:N]

    return (nodes_update, edges_update, receivers, senders, glat,
            node_graph_idx, edge_graph_idx)


# speedup vs baseline: 4.0465x; 1.0107x over previous
import functools

import jax
import jax.numpy as jnp
from jax import lax
from jax.experimental import pallas as pl
from jax.experimental.pallas import tpu as pltpu
from jax.experimental.pallas import tpu_sc as plsc

N = 10000
E = 320000
D = 128
B = 256
AV = 128
BV = 16
NA = 9
NB = 3

NC = 2
NS = 16
NW = NC * NS

EG = 128
N_EGROUPS = E // EG
N_NGROUPS = -(-N // EG)
NPAD = N_NGROUPS * EG


def _prep_body(at_ref, bt_ref, we_ref, be_ref, wn_ref, bn_ref, gt_ref,
               e0_ref, e1_ref, e2_ref, nt_ref,
               taw_ref, tbc_ref, glat_ref, cidx_ref, nidx_ref):
    wn = wn_ref[...]
    bn = bn_ref[...]
    for f in range(NA):
        r = jnp.dot(at_ref[f], wn, preferred_element_type=jnp.float32)
        if f == 0:
            r = r + bn
        taw_ref[pl.ds(f * AV, AV), :] = r

    we = we_ref[...]
    be = be_ref[...]
    t0 = jnp.dot(bt_ref[0], we, preferred_element_type=jnp.float32) + be
    t1 = jnp.dot(bt_ref[1], we, preferred_element_type=jnp.float32)
    t2 = jnp.dot(bt_ref[2], we, preferred_element_type=jnp.float32)
    for i1 in range(BV):
        t01 = t0 + t1[i1:i1 + 1, :]
        for i2 in range(BV):
            tbc_ref[pl.ds(i2 * 256 + i1 * 16, BV), :] = t01 + t2[i2:i2 + 1, :]

    glat_ref[...] = jnp.broadcast_to(gt_ref[...], (B, D))
    cidx_ref[...] = e0_ref[...] + 16 * e1_ref[...] + 256 * e2_ref[...]
    fld = lax.broadcasted_iota(jnp.int32, (NA * N_NGROUPS, D), 0) // N_NGROUPS
    nidx_ref[...] = nt_ref[...] + AV * fld


def _relu_rows(rows_v):
    def relu_row(r2, c):
        for rr in range(2):
            r = r2 * 2 + rr
            for j in range(D // 16):
                sl = pl.ds(j * 16, 16)
                rows_v[r, sl] = jnp.maximum(rows_v[r, sl], 0.0)
        return c

    lax.fori_loop(0, EG // 2, relu_row, 0)


NEB = 4


def _sc_body(tbc, taw, cidx, nidx, eout, nout,
             ei0, ei1, ei2, ei3, er0, er1, er2, er3,
             tbc_sh, taw_sh,
             g0, g1, g2, g3, w0, w1, w2, w3):
    cid = lax.axis_index("c")
    sid = lax.axis_index("s")
    wid = sid * NC + cid
    idx_b = [ei0, ei1, ei2, ei3]
    row_b = [er0, er1, er2, er3]
    gsem = [g0, g1, g2, g3]
    wsem = [w0, w1, w2, w3]

    @pl.when(sid == 0)
    def _stage():
        pltpu.sync_copy(tbc, tbc_sh)
        pltpu.sync_copy(taw, taw_sh)

    plsc.subcore_barrier()

    my_eg = (N_EGROUPS - wid + NW - 1) // NW

    def edge_blk(i, carry):
        descs = []
        for k in range(NEB):
            @pl.when(i > 0)
            def _drain(_k=k):
                pltpu.make_async_copy(row_b[_k], eout.at[pl.ds(0, EG)],
                                      wsem[_k]).wait()

            base = (wid + (i * NEB + k) * NW) * EG
            pltpu.sync_copy(cidx.at[pl.ds(base, EG)], idx_b[k])
            descs.append(pltpu.async_copy(tbc_sh.at[idx_b[k]], row_b[k],
                                          gsem[k]))
        for k in range(NEB):
            base = (wid + (i * NEB + k) * NW) * EG
            descs[k].wait()
            _relu_rows(row_b[k])
            pltpu.async_copy(row_b[k], eout.at[pl.ds(base, EG)], wsem[k])
        return carry

    nblk = my_eg // NEB
    lax.fori_loop(0, nblk, edge_blk, 0)
    for k in range(NEB):
        @pl.when(nblk > 0)
        def _drain_tail(_k=k):
            pltpu.make_async_copy(row_b[_k], eout.at[pl.ds(0, EG)],
                                  wsem[_k]).wait()

    def edge_one(base):
        pltpu.sync_copy(cidx.at[pl.ds(base, EG)], idx_b[0])
        pltpu.async_copy(tbc_sh.at[idx_b[0]], row_b[0], gsem[0]).wait()
        _relu_rows(row_b[0])
        pltpu.sync_copy(row_b[0], eout.at[pl.ds(base, EG)])

    rem = my_eg - nblk * NEB
    for t in range(1, NEB):
        @pl.when(rem >= t)
        def _tail(_t=t):
            edge_one((wid + (nblk * NEB + _t - 1) * NW) * EG)

    def node_group(i, carry):
        g = wid + i * NW
        pltpu.sync_copy(nidx.at[pl.ds(g * EG, EG)], idx_b[0])
        d0 = pltpu.async_copy(taw_sh.at[idx_b[0]], row_b[0], gsem[0])
        pltpu.sync_copy(nidx.at[pl.ds((N_NGROUPS + g) * EG, EG)], idx_b[1])
        dprev = pltpu.async_copy(taw_sh.at[idx_b[1]], row_b[1], gsem[1])
        d0.wait()
        buf = [row_b[1], row_b[2]]
        ibuf = [idx_b[1], idx_b[2]]
        bsem = [gsem[1], gsem[2]]
        for f in range(2, NA + 1):
            pb = (f - 2) % 2
            if f < NA:
                cb = (f - 1) % 2
                pltpu.sync_copy(nidx.at[pl.ds((f * N_NGROUPS + g) * EG, EG)],
                                ibuf[cb])
                dcur = pltpu.async_copy(taw_sh.at[ibuf[cb]], buf[cb], bsem[cb])
            dprev.wait()

            def add_row(r, c, _b=pb):
                for j in range(D // 16):
                    sl = pl.ds(j * 16, 16)
                    row_b[0][r, sl] = row_b[0][r, sl] + buf[_b][r, sl]
                return c

            lax.fori_loop(0, EG, add_row, 0)
            if f < NA:
                dprev = dcur
        _relu_rows(row_b[0])
        pltpu.sync_copy(row_b[0], nout.at[pl.ds(g * EG, EG)])
        return carry

    my_ng = (N_NGROUPS - wid + NW - 1) // NW
    lax.fori_loop(0, my_ng, node_group, 0)


def kernel(nodes, edges, receivers, senders, node_graph_idx, edge_graph_idx,
           atom_tables, bond_tables, W_edge, b_edge, W_node, b_node,
           global_table):
    e0 = edges[:, 0].reshape(E // D, D)
    e1 = edges[:, 1].reshape(E // D, D)
    e2 = edges[:, 2].reshape(E // D, D)
    nodes_t = jnp.pad(nodes.T, ((0, 0), (0, NPAD - N))).reshape(
        NA * N_NGROUPS, EG)

    taw, tbc, glat, cidx2d, nidx2d = pl.pallas_call(
        _prep_body,
        out_shape=(
            jax.ShapeDtypeStruct((NA * AV, D), jnp.float32),
            jax.ShapeDtypeStruct((BV * BV * BV, D), jnp.float32),
            jax.ShapeDtypeStruct((B, D), jnp.float32),
            jax.ShapeDtypeStruct((E // D, D), jnp.int32),
            jax.ShapeDtypeStruct((NA * N_NGROUPS, EG), jnp.int32),
        ),
    )(atom_tables, bond_tables, W_edge, b_edge.reshape(1, D),
      W_node, b_node.reshape(1, D), global_table,
      e0, e1, e2, nodes_t)

    cidx = cidx2d.reshape(E)
    nidx = nidx2d.reshape(NA * NPAD)

    mesh = plsc.VectorSubcoreMesh(core_axis_name="c", subcore_axis_name="s",
                                  num_cores=NC, num_subcores=NS)
    sc = functools.partial(
        pl.kernel,
        out_type=(
            jax.ShapeDtypeStruct((E, D), jnp.float32),
            jax.ShapeDtypeStruct((NPAD, D), jnp.float32),
        ),
        mesh=mesh,
        scratch_types=(
            [pltpu.VMEM((EG,), jnp.int32)] * 4
            + [pltpu.VMEM((EG, D), jnp.float32)] * 4
            + [pltpu.VMEM_SHARED((BV * BV * BV, D), jnp.float32)]
            + [pltpu.VMEM_SHARED((NA * AV, D), jnp.float32)]
            + [pltpu.SemaphoreType.DMA] * 8
        ),
    )(_sc_body)

    edges_update, nodes_pad = sc(tbc, taw, cidx, nidx)
    nodes_update = nodes_pad[:N]

    return (nodes_update, edges_update, receivers, senders, glat,
            node_graph_idx, edge_graph_idx)


# speedup vs baseline: 4.2506x; 1.0505x over previous
import functools

import jax
import jax.numpy as jnp
from jax import lax
from jax.experimental import pallas as pl
from jax.experimental.pallas import tpu as pltpu
from jax.experimental.pallas import tpu_sc as plsc

N = 10000
E = 320000
D = 128
B = 256
AV = 128
BV = 16
NA = 9
NB = 3

NC = 2
NS = 16
NW = NC * NS

EG = 128
N_EGROUPS = E // EG
N_NGROUPS = -(-N // EG)
NPAD = N_NGROUPS * EG
EPW = -(-N_EGROUPS // NW)


def _prep_body(at_ref, bt_ref, we_ref, be_ref, wn_ref, bn_ref, gt_ref,
               e0_ref, e1_ref, e2_ref, nt_ref,
               taw_ref, tbc_ref, glat_ref, cidx_ref, nidx_ref):
    wn = wn_ref[...]
    bn = bn_ref[...]
    for f in range(NA):
        r = jnp.dot(at_ref[f], wn, preferred_element_type=jnp.float32)
        if f == 0:
            r = r + bn
        taw_ref[pl.ds(f * AV, AV), :] = r

    we = we_ref[...]
    be = be_ref[...]
    t0 = jnp.dot(bt_ref[0], we, preferred_element_type=jnp.float32) + be
    t1 = jnp.dot(bt_ref[1], we, preferred_element_type=jnp.float32)
    t2 = jnp.dot(bt_ref[2], we, preferred_element_type=jnp.float32)
    for i1 in range(BV):
        t01 = t0 + t1[i1:i1 + 1, :]
        for i2 in range(BV):
            tbc_ref[pl.ds(i2 * 256 + i1 * 16, BV), :] = t01 + t2[i2:i2 + 1, :]

    glat_ref[...] = jnp.broadcast_to(gt_ref[...], (B, D))
    cidx_ref[...] = e0_ref[...] + 16 * e1_ref[...] + 256 * e2_ref[...]
    fld = lax.broadcasted_iota(jnp.int32, (NA * N_NGROUPS, D), 0) // N_NGROUPS
    nidx_ref[...] = nt_ref[...] + AV * fld


def _relu_rows(rows_v):
    def relu_row(r2, c):
        for rr in range(2):
            r = r2 * 2 + rr
            for j in range(D // 16):
                sl = pl.ds(j * 16, 16)
                rows_v[r, sl] = jnp.maximum(rows_v[r, sl], 0.0)
        return c

    lax.fori_loop(0, EG // 2, relu_row, 0)


NEB = 4


def _sc_body(tbc, taw, cidx, nidx, eout, nout,
             eslab, nslab, er0, er1, er2, er3,
             tbc_sh, taw_sh,
             g0, g1, g2, g3, w0, w1, w2, w3):
    cid = lax.axis_index("c")
    sid = lax.axis_index("s")
    wid = sid * NC + cid
    row_b = [er0, er1, er2, er3]
    gsem = [g0, g1, g2, g3]
    wsem = [w0, w1, w2, w3]

    @pl.when(sid == 0)
    def _stage():
        pltpu.sync_copy(tbc, tbc_sh)
        pltpu.sync_copy(taw, taw_sh)

    plsc.subcore_barrier()

    my_eg = (N_EGROUPS - wid + NW - 1) // NW
    pltpu.sync_copy(cidx.at[pl.ds(wid * (EPW * EG), EPW * EG)], eslab)

    def edge_blk(i, carry):
        descs = []
        for k in range(NEB):
            @pl.when(i > 0)
            def _drain(_k=k):
                pltpu.make_async_copy(row_b[_k], eout.at[pl.ds(0, EG)],
                                      wsem[_k]).wait()

            j = i * NEB + k
            descs.append(pltpu.async_copy(
                tbc_sh.at[eslab.at[pl.ds(j * EG, EG)]], row_b[k], gsem[k]))
        for k in range(NEB):
            base = (wid + (i * NEB + k) * NW) * EG
            descs[k].wait()
            _relu_rows(row_b[k])
            pltpu.async_copy(row_b[k], eout.at[pl.ds(base, EG)], wsem[k])
        return carry

    nblk = my_eg // NEB
    lax.fori_loop(0, nblk, edge_blk, 0)
    for k in range(NEB):
        @pl.when(nblk > 0)
        def _drain_tail(_k=k):
            pltpu.make_async_copy(row_b[_k], eout.at[pl.ds(0, EG)],
                                  wsem[_k]).wait()

    rem = my_eg - nblk * NEB
    for t in range(1, NEB):
        @pl.when(rem >= t)
        def _tail(_t=t):
            j = nblk * NEB + _t - 1
            base = (wid + j * NW) * EG
            pltpu.async_copy(tbc_sh.at[eslab.at[pl.ds(j * EG, EG)]],
                             row_b[0], gsem[0]).wait()
            _relu_rows(row_b[0])
            pltpu.sync_copy(row_b[0], eout.at[pl.ds(base, EG)])

    def node_group(i, carry):
        g = wid + i * NW
        pltpu.sync_copy(nidx.at[pl.ds(g * (NA * EG), NA * EG)], nslab)
        d0 = pltpu.async_copy(taw_sh.at[nslab.at[pl.ds(0, EG)]],
                              row_b[0], gsem[0])
        dprev = pltpu.async_copy(taw_sh.at[nslab.at[pl.ds(EG, EG)]],
                                 row_b[1], gsem[1])
        d0.wait()
        buf = [row_b[1], row_b[2]]
        bsem = [gsem[1], gsem[2]]
        for f in range(2, NA + 1):
            pb = (f - 2) % 2
            if f < NA:
                cb = (f - 1) % 2
                dcur = pltpu.async_copy(
                    taw_sh.at[nslab.at[pl.ds(f * EG, EG)]], buf[cb], bsem[cb])
            dprev.wait()

            def add_row(r, c, _b=pb):
                for j in range(D // 16):
                    sl = pl.ds(j * 16, 16)
                    row_b[0][r, sl] = row_b[0][r, sl] + buf[_b][r, sl]
                return c

            lax.fori_loop(0, EG, add_row, 0)
            if f < NA:
                dprev = dcur
        _relu_rows(row_b[0])

        @pl.when(g < N_NGROUPS - 1)
        def _full():
            pltpu.sync_copy(row_b[0], nout.at[pl.ds(g * EG, EG)])

        @pl.when(g == N_NGROUPS - 1)
        def _tail16():
            pltpu.sync_copy(row_b[0].at[pl.ds(0, N - (N_NGROUPS - 1) * EG)],
                            nout.at[pl.ds((N_NGROUPS - 1) * EG,
                                          N - (N_NGROUPS - 1) * EG)])
        return carry

    my_ng = (N_NGROUPS - wid + NW - 1) // NW
    lax.fori_loop(0, my_ng, node_group, 0)


def kernel(nodes, edges, receivers, senders, node_graph_idx, edge_graph_idx,
           atom_tables, bond_tables, W_edge, b_edge, W_node, b_node,
           global_table):
    e0 = edges[:, 0].reshape(E // D, D)
    e1 = edges[:, 1].reshape(E // D, D)
    e2 = edges[:, 2].reshape(E // D, D)
    nodes_t = jnp.pad(nodes.T, ((0, 0), (0, NPAD - N))).reshape(
        NA * N_NGROUPS, EG)

    taw, tbc, glat, cidx2d, nidx2d = pl.pallas_call(
        _prep_body,
        out_shape=(
            jax.ShapeDtypeStruct((NA * AV, D), jnp.float32),
            jax.ShapeDtypeStruct((BV * BV * BV, D), jnp.float32),
            jax.ShapeDtypeStruct((B, D), jnp.float32),
            jax.ShapeDtypeStruct((E // D, D), jnp.int32),
            jax.ShapeDtypeStruct((NA * N_NGROUPS, EG), jnp.int32),
        ),
    )(atom_tables, bond_tables, W_edge, b_edge.reshape(1, D),
      W_node, b_node.reshape(1, D), global_table,
      e0, e1, e2, nodes_t)

    cidx_wm = jnp.pad(cidx2d, ((0, EPW * NW - N_EGROUPS), (0, 0)))
    cidx = cidx_wm.reshape(EPW, NW, EG).transpose(1, 0, 2).reshape(
        EPW * NW * EG)
    nidx = nidx2d.reshape(NA, N_NGROUPS, EG).transpose(1, 0, 2).reshape(
        NA * NPAD)

    mesh = plsc.VectorSubcoreMesh(core_axis_name="c", subcore_axis_name="s",
                                  num_cores=NC, num_subcores=NS)
    sc = functools.partial(
        pl.kernel,
        out_type=(
            jax.ShapeDtypeStruct((E, D), jnp.float32),
            jax.ShapeDtypeStruct((N, D), jnp.float32),
        ),
        mesh=mesh,
        scratch_types=(
            [pltpu.VMEM((EPW * EG,), jnp.int32)]
            + [pltpu.VMEM((NA * EG,), jnp.int32)]
            + [pltpu.VMEM((EG, D), jnp.float32)] * 4
            + [pltpu.VMEM_SHARED((BV * BV * BV, D), jnp.float32)]
            + [pltpu.VMEM_SHARED((NA * AV, D), jnp.float32)]
            + [pltpu.SemaphoreType.DMA] * 8
        ),
    )(_sc_body)

    edges_update, nodes_update = sc(tbc, taw, cidx, nidx)

    return (nodes_update, edges_update, receivers, senders, glat,
            node_graph_idx, edge_graph_idx)


# speedup vs baseline: 4.5744x; 1.0762x over previous
import functools

import jax
import jax.numpy as jnp
from jax import lax
from jax.experimental import pallas as pl
from jax.experimental.pallas import tpu as pltpu
from jax.experimental.pallas import tpu_sc as plsc

N = 10000
E = 320000
D = 128
B = 256
AV = 128
BV = 16
NA = 9
NB = 3

NC = 2
NS = 16
NW = NC * NS

EG = 128
N_EGROUPS = E // EG
N_NGROUPS = -(-N // EG)
NPAD = N_NGROUPS * EG
EPW = -(-N_EGROUPS // NW)


def _prep_body(at_ref, bt_ref, we_ref, be_ref, wn_ref, bn_ref, gt_ref,
               e0_ref, e1_ref, e2_ref, nt_ref,
               taw_ref, tbc_ref, glat_ref, cidx_ref, nidx_ref):
    wn = wn_ref[...]
    bn = bn_ref[...]
    for f in range(NA):
        r = jnp.dot(at_ref[f], wn, preferred_element_type=jnp.float32)
        if f == 0:
            r = r + bn
        taw_ref[pl.ds(f * AV, AV), :] = r

    we = we_ref[...]
    be = be_ref[...]
    t0 = jnp.dot(bt_ref[0], we, preferred_element_type=jnp.float32) + be
    t1 = jnp.dot(bt_ref[1], we, preferred_element_type=jnp.float32)
    t2 = jnp.dot(bt_ref[2], we, preferred_element_type=jnp.float32)
    for i1 in range(BV):
        t01 = t0 + t1[i1:i1 + 1, :]
        for i2 in range(BV):
            tbc_ref[pl.ds(i2 * 256 + i1 * 16, BV), :] = t01 + t2[i2:i2 + 1, :]

    glat_ref[...] = jnp.broadcast_to(gt_ref[...], (B, D))
    cidx_ref[...] = e0_ref[...] + 16 * e1_ref[...] + 256 * e2_ref[...]
    fld = lax.broadcasted_iota(jnp.int32, (NA * N_NGROUPS, D), 0) // N_NGROUPS
    nidx_ref[...] = nt_ref[...] + AV * fld


def _relu_rows(rows_v):
    def relu_row(r2, c):
        for rr in range(2):
            r = r2 * 2 + rr
            for j in range(D // 16):
                sl = pl.ds(j * 16, 16)
                rows_v[r, sl] = jnp.maximum(rows_v[r, sl], 0.0)
        return c

    lax.fori_loop(0, EG // 2, relu_row, 0)


NEB = 4


def _sc_body(tbc, taw, cidx, nidx, eout, nout,
             eslab, nslab, er0, er1, er2, er3,
             tbc_sh, taw_sh,
             g0, g1, g2, g3, w0, w1, w2, w3):
    cid = lax.axis_index("c")
    sid = lax.axis_index("s")
    wid = sid * NC + cid
    row_b = [er0, er1, er2, er3]
    gsem = [g0, g1, g2, g3]
    wsem = [w0, w1, w2, w3]

    @pl.when(sid == 0)
    def _stage():
        pltpu.sync_copy(tbc, tbc_sh)
        pltpu.sync_copy(taw, taw_sh)

    plsc.subcore_barrier()

    my_eg = (N_EGROUPS - wid + NW - 1) // NW
    pltpu.sync_copy(cidx.at[pl.ds(wid * (EPW * EG), EPW * EG)], eslab)

    def edge_blk(i, carry):
        descs = []
        for k in range(NEB):
            @pl.when(i > 0)
            def _drain(_k=k):
                pltpu.make_async_copy(row_b[_k], eout.at[pl.ds(0, EG)],
                                      wsem[_k]).wait()

            j = i * NEB + k
            descs.append(pltpu.async_copy(
                tbc_sh.at[eslab.at[pl.ds(j * EG, EG)]], row_b[k], gsem[k]))
        for k in range(NEB):
            base = (wid + (i * NEB + k) * NW) * EG
            descs[k].wait()
            _relu_rows(row_b[k])
            pltpu.async_copy(row_b[k], eout.at[pl.ds(base, EG)], wsem[k])
        return carry

    nblk = my_eg // NEB
    lax.fori_loop(0, nblk, edge_blk, 0)
    for k in range(NEB):
        @pl.when(nblk > 0)
        def _drain_tail(_k=k):
            pltpu.make_async_copy(row_b[_k], eout.at[pl.ds(0, EG)],
                                  wsem[_k]).wait()

    rem = my_eg - nblk * NEB
    for t in range(1, NEB):
        @pl.when(rem >= t)
        def _tail(_t=t):
            j = nblk * NEB + _t - 1
            base = (wid + j * NW) * EG
            pltpu.async_copy(tbc_sh.at[eslab.at[pl.ds(j * EG, EG)]],
                             row_b[0], gsem[0]).wait()
            _relu_rows(row_b[0])
            pltpu.sync_copy(row_b[0], eout.at[pl.ds(base, EG)])

    def node_group(i, carry):
        g = wid + i * NW
        pltpu.sync_copy(nidx.at[pl.ds(g * (NA * EG), NA * EG)], nslab)
        pltpu.async_copy(taw_sh.at[nslab.at[pl.ds(0, EG)]],
                         row_b[0], gsem[0]).wait()
        for f in range(1, NA):
            pltpu.async_copy(taw_sh.at[nslab.at[pl.ds(f * EG, EG)]],
                             row_b[0], gsem[0], add=True).wait()
        _relu_rows(row_b[0])

        @pl.when(g < N_NGROUPS - 1)
        def _full():
            pltpu.sync_copy(row_b[0], nout.at[pl.ds(g * EG, EG)])

        @pl.when(g == N_NGROUPS - 1)
        def _tail16():
            pltpu.sync_copy(row_b[0].at[pl.ds(0, N - (N_NGROUPS - 1) * EG)],
                            nout.at[pl.ds((N_NGROUPS - 1) * EG,
                                          N - (N_NGROUPS - 1) * EG)])
        return carry

    my_ng = (N_NGROUPS - wid + NW - 1) // NW
    lax.fori_loop(0, my_ng, node_group, 0)


def kernel(nodes, edges, receivers, senders, node_graph_idx, edge_graph_idx,
           atom_tables, bond_tables, W_edge, b_edge, W_node, b_node,
           global_table):
    e0 = edges[:, 0].reshape(E // D, D)
    e1 = edges[:, 1].reshape(E // D, D)
    e2 = edges[:, 2].reshape(E // D, D)
    nodes_t = jnp.pad(nodes.T, ((0, 0), (0, NPAD - N))).reshape(
        NA * N_NGROUPS, EG)

    taw, tbc, glat, cidx2d, nidx2d = pl.pallas_call(
        _prep_body,
        out_shape=(
            jax.ShapeDtypeStruct((NA * AV, D), jnp.float32),
            jax.ShapeDtypeStruct((BV * BV * BV, D), jnp.float32),
            jax.ShapeDtypeStruct((B, D), jnp.float32),
            jax.ShapeDtypeStruct((E // D, D), jnp.int32),
            jax.ShapeDtypeStruct((NA * N_NGROUPS, EG), jnp.int32),
        ),
    )(atom_tables, bond_tables, W_edge, b_edge.reshape(1, D),
      W_node, b_node.reshape(1, D), global_table,
      e0, e1, e2, nodes_t)

    cidx_wm = jnp.pad(cidx2d, ((0, EPW * NW - N_EGROUPS), (0, 0)))
    cidx = cidx_wm.reshape(EPW, NW, EG).transpose(1, 0, 2).reshape(
        EPW * NW * EG)
    nidx = nidx2d.reshape(NA, N_NGROUPS, EG).transpose(1, 0, 2).reshape(
        NA * NPAD)

    mesh = plsc.VectorSubcoreMesh(core_axis_name="c", subcore_axis_name="s",
                                  num_cores=NC, num_subcores=NS)
    sc = functools.partial(
        pl.kernel,
        out_type=(
            jax.ShapeDtypeStruct((E, D), jnp.float32),
            jax.ShapeDtypeStruct((N, D), jnp.float32),
        ),
        mesh=mesh,
        scratch_types=(
            [pltpu.VMEM((EPW * EG,), jnp.int32)]
            + [pltpu.VMEM((NA * EG,), jnp.int32)]
            + [pltpu.VMEM((EG, D), jnp.float32)] * 4
            + [pltpu.VMEM_SHARED((BV * BV * BV, D), jnp.float32)]
            + [pltpu.VMEM_SHARED((NA * AV, D), jnp.float32)]
            + [pltpu.SemaphoreType.DMA] * 8
        ),
    )(_sc_body)

    edges_update, nodes_update = sc(tbc, taw, cidx, nidx)

    return (nodes_update, edges_update, receivers, senders, glat,
            node_graph_idx, edge_graph_idx)


# speedup vs baseline: 4.6251x; 1.0111x over previous
import functools

import jax
import jax.numpy as jnp
from jax import lax
from jax.experimental import pallas as pl
from jax.experimental.pallas import tpu as pltpu
from jax.experimental.pallas import tpu_sc as plsc

N = 10000
E = 320000
D = 128
B = 256
AV = 128
BV = 16
NA = 9
NB = 3

NC = 2
NS = 16
NW = NC * NS

EG = 128
N_EGROUPS = E // EG
N_NGROUPS = -(-N // EG)
NPAD = N_NGROUPS * EG
EPW = -(-N_EGROUPS // NW)


def _prep_body(at_ref, bt_ref, we_ref, be_ref, wn_ref, bn_ref, gt_ref,
               e0_ref, e1_ref, e2_ref, nt_ref,
               taw_ref, tbc_ref, glat_ref, cidx_ref, nidx_ref):
    wn = wn_ref[...]
    bn = bn_ref[...]
    for f in range(NA):
        r = jnp.dot(at_ref[f], wn, preferred_element_type=jnp.float32)
        if f == 0:
            r = r + bn
        taw_ref[pl.ds(f * AV, AV), :] = r

    we = we_ref[...]
    be = be_ref[...]
    t0 = jnp.dot(bt_ref[0], we, preferred_element_type=jnp.float32) + be
    t1 = jnp.dot(bt_ref[1], we, preferred_element_type=jnp.float32)
    t2 = jnp.dot(bt_ref[2], we, preferred_element_type=jnp.float32)
    for i1 in range(BV):
        t01 = t0 + t1[i1:i1 + 1, :]
        for i2 in range(BV):
            tbc_ref[pl.ds(i2 * 256 + i1 * 16, BV), :] = t01 + t2[i2:i2 + 1, :]

    glat_ref[...] = jnp.broadcast_to(gt_ref[...], (B, D))
    cidx_ref[...] = e0_ref[...] + 16 * e1_ref[...] + 256 * e2_ref[...]
    fld = lax.broadcasted_iota(jnp.int32, (NA * N_NGROUPS, D), 0) // N_NGROUPS
    nidx_ref[...] = nt_ref[...] + AV * fld


def _relu_rows(rows_v):
    def relu_row(r2, c):
        for rr in range(2):
            r = r2 * 2 + rr
            for j in range(D // 16):
                sl = pl.ds(j * 16, 16)
                rows_v[r, sl] = jnp.maximum(rows_v[r, sl], 0.0)
        return c

    lax.fori_loop(0, EG // 2, relu_row, 0)


NEB = 4


def _sc_body(tbc, taw, cidx, nidx, eout, nout,
             eslab, nslab, er0, er1, er2, er3,
             tbc_sh, taw_sh,
             g0, g1, g2, g3, w0, w1, w2, w3):
    cid = lax.axis_index("c")
    sid = lax.axis_index("s")
    wid = sid * NC + cid
    row_b = [er0, er1, er2, er3]
    gsem = [g0, g1, g2, g3]
    wsem = [w0, w1, w2, w3]

    @pl.when(sid == 0)
    def _stage():
        pltpu.sync_copy(tbc, tbc_sh)
        pltpu.sync_copy(taw, taw_sh)

    plsc.subcore_barrier()

    my_eg = (N_EGROUPS - wid + NW - 1) // NW
    pltpu.sync_copy(cidx.at[pl.ds(wid * (EPW * EG), EPW * EG)], eslab)

    def edge_blk(i, carry):
        descs = []
        for k in range(NEB):
            @pl.when(i > 0)
            def _drain(_k=k):
                pltpu.make_async_copy(row_b[_k], eout.at[pl.ds(0, EG)],
                                      wsem[_k]).wait()

            j = i * NEB + k
            descs.append(pltpu.async_copy(
                tbc_sh.at[eslab.at[pl.ds(j * EG, EG)]], row_b[k], gsem[k]))
        for k in range(NEB):
            base = (wid + (i * NEB + k) * NW) * EG
            descs[k].wait()
            _relu_rows(row_b[k])
            pltpu.async_copy(row_b[k], eout.at[pl.ds(base, EG)], wsem[k])
        return carry

    nblk = my_eg // NEB
    lax.fori_loop(0, nblk, edge_blk, 0)
    for k in range(NEB):
        @pl.when(nblk > 0)
        def _drain_tail(_k=k):
            pltpu.make_async_copy(row_b[_k], eout.at[pl.ds(0, EG)],
                                  wsem[_k]).wait()

    rem = my_eg - nblk * NEB
    for t in range(1, NEB):
        @pl.when(rem >= t)
        def _tail(_t=t):
            j = nblk * NEB + _t - 1
            base = (wid + j * NW) * EG
            pltpu.async_copy(tbc_sh.at[eslab.at[pl.ds(j * EG, EG)]],
                             row_b[0], gsem[0]).wait()
            _relu_rows(row_b[0])
            pltpu.sync_copy(row_b[0], eout.at[pl.ds(base, EG)])

    def node_group(i, carry):
        g = wid + i * NW
        pltpu.sync_copy(nidx.at[pl.ds(g * (NA * EG), NA * EG)], nslab)
        descs = [pltpu.async_copy(taw_sh.at[nslab.at[pl.ds(0, EG)]],
                                  row_b[0], gsem[0])]
        for f in range(1, NA):
            descs.append(pltpu.async_copy(taw_sh.at[nslab.at[pl.ds(f * EG, EG)]],
                                          row_b[0], gsem[0], add=True))
        for d in descs:
            d.wait()
        _relu_rows(row_b[0])

        @pl.when(g < N_NGROUPS - 1)
        def _full():
            pltpu.sync_copy(row_b[0], nout.at[pl.ds(g * EG, EG)])

        @pl.when(g == N_NGROUPS - 1)
        def _tail16():
            pltpu.sync_copy(row_b[0].at[pl.ds(0, N - (N_NGROUPS - 1) * EG)],
                            nout.at[pl.ds((N_NGROUPS - 1) * EG,
                                          N - (N_NGROUPS - 1) * EG)])
        return carry

    my_ng = (N_NGROUPS - wid + NW - 1) // NW
    lax.fori_loop(0, my_ng, node_group, 0)


def kernel(nodes, edges, receivers, senders, node_graph_idx, edge_graph_idx,
           atom_tables, bond_tables, W_edge, b_edge, W_node, b_node,
           global_table):
    e0 = edges[:, 0].reshape(E // D, D)
    e1 = edges[:, 1].reshape(E // D, D)
    e2 = edges[:, 2].reshape(E // D, D)
    nodes_t = jnp.pad(nodes.T, ((0, 0), (0, NPAD - N))).reshape(
        NA * N_NGROUPS, EG)

    taw, tbc, glat, cidx2d, nidx2d = pl.pallas_call(
        _prep_body,
        out_shape=(
            jax.ShapeDtypeStruct((NA * AV, D), jnp.float32),
            jax.ShapeDtypeStruct((BV * BV * BV, D), jnp.float32),
            jax.ShapeDtypeStruct((B, D), jnp.float32),
            jax.ShapeDtypeStruct((E // D, D), jnp.int32),
            jax.ShapeDtypeStruct((NA * N_NGROUPS, EG), jnp.int32),
        ),
    )(atom_tables, bond_tables, W_edge, b_edge.reshape(1, D),
      W_node, b_node.reshape(1, D), global_table,
      e0, e1, e2, nodes_t)

    cidx_wm = jnp.pad(cidx2d, ((0, EPW * NW - N_EGROUPS), (0, 0)))
    cidx = cidx_wm.reshape(EPW, NW, EG).transpose(1, 0, 2).reshape(
        EPW * NW * EG)
    nidx = nidx2d.reshape(NA, N_NGROUPS, EG).transpose(1, 0, 2).reshape(
        NA * NPAD)

    mesh = plsc.VectorSubcoreMesh(core_axis_name="c", subcore_axis_name="s",
                                  num_cores=NC, num_subcores=NS)
    sc = functools.partial(
        pl.kernel,
        out_type=(
            jax.ShapeDtypeStruct((E, D), jnp.float32),
            jax.ShapeDtypeStruct((N, D), jnp.float32),
        ),
        mesh=mesh,
        scratch_types=(
            [pltpu.VMEM((EPW * EG,), jnp.int32)]
            + [pltpu.VMEM((NA * EG,), jnp.int32)]
            + [pltpu.VMEM((EG, D), jnp.float32)] * 4
            + [pltpu.VMEM_SHARED((BV * BV * BV, D), jnp.float32)]
            + [pltpu.VMEM_SHARED((NA * AV, D), jnp.float32)]
            + [pltpu.SemaphoreType.DMA] * 8
        ),
    )(_sc_body)

    edges_update, nodes_update = sc(tbc, taw, cidx, nidx)

    return (nodes_update, edges_update, receivers, senders, glat,
            node_graph_idx, edge_graph_idx)


# speedup vs baseline: 4.6256x; 1.0001x over previous
import functools

import jax
import jax.numpy as jnp
from jax import lax
from jax.experimental import pallas as pl
from jax.experimental.pallas import tpu as pltpu
from jax.experimental.pallas import tpu_sc as plsc

N = 10000
E = 320000
D = 128
B = 256
AV = 128
BV = 16
NA = 9
NB = 3

NC = 2
NS = 16
NW = NC * NS

EG = 128
N_EGROUPS = E // EG
N_NGROUPS = -(-N // EG)
NPAD = N_NGROUPS * EG
EPW = -(-N_EGROUPS // NW)


def _prep_body(at_ref, bt_ref, we_ref, be_ref, wn_ref, bn_ref, gt_ref,
               e0_ref, e1_ref, e2_ref, nt_ref,
               taw_ref, tbc_ref, glat_ref, cidx_ref, nidx_ref):
    wn = wn_ref[...]
    bn = bn_ref[...]
    for f in range(NA):
        r = jnp.dot(at_ref[f], wn, preferred_element_type=jnp.float32)
        if f == 0:
            r = r + bn
        taw_ref[pl.ds(f * AV, AV), :] = r

    we = we_ref[...]
    be = be_ref[...]
    t0 = jnp.dot(bt_ref[0], we, preferred_element_type=jnp.float32) + be
    t1 = jnp.dot(bt_ref[1], we, preferred_element_type=jnp.float32)
    t2 = jnp.dot(bt_ref[2], we, preferred_element_type=jnp.float32)
    for i1 in range(BV):
        t01 = t0 + t1[i1:i1 + 1, :]
        for i2 in range(BV):
            tbc_ref[pl.ds(i2 * 256 + i1 * 16, BV), :] = t01 + t2[i2:i2 + 1, :]

    glat_ref[...] = jnp.broadcast_to(gt_ref[...], (B, D))
    cidx_ref[...] = e0_ref[...] + 16 * e1_ref[...] + 256 * e2_ref[...]
    fld = lax.broadcasted_iota(jnp.int32, (NA * N_NGROUPS, D), 0) // N_NGROUPS
    nidx_ref[...] = nt_ref[...] + AV * fld


def _relu_rows(rows_v):
    def relu_row(r2, c):
        for rr in range(2):
            r = r2 * 2 + rr
            for j in range(D // 16):
                sl = pl.ds(j * 16, 16)
                rows_v[r, sl] = jnp.maximum(rows_v[r, sl], 0.0)
        return c

    lax.fori_loop(0, EG // 2, relu_row, 0)


NEB = 4


def _sc_body(tbc, taw, cidx, nidx, eout, nout,
             eslab, nslab, nslab2, er0, er1, er2, er3,
             tbc_sh, taw_sh,
             g0, g1, g2, g3, w0, w1, w2, w3):
    cid = lax.axis_index("c")
    sid = lax.axis_index("s")
    wid = sid * NC + cid
    row_b = [er0, er1, er2, er3]
    gsem = [g0, g1, g2, g3]
    wsem = [w0, w1, w2, w3]

    @pl.when(sid == 0)
    def _stage():
        pltpu.sync_copy(tbc, tbc_sh)
        pltpu.sync_copy(taw, taw_sh)

    plsc.subcore_barrier()

    my_eg = (N_EGROUPS - wid + NW - 1) // NW
    pltpu.sync_copy(cidx.at[pl.ds(wid * (EPW * EG), EPW * EG)], eslab)

    def edge_blk(i, carry):
        descs = []
        for k in range(NEB):
            @pl.when(i > 0)
            def _drain(_k=k):
                pltpu.make_async_copy(row_b[_k], eout.at[pl.ds(0, EG)],
                                      wsem[_k]).wait()

            j = i * NEB + k
            descs.append(pltpu.async_copy(
                tbc_sh.at[eslab.at[pl.ds(j * EG, EG)]], row_b[k], gsem[k]))
        for k in range(NEB):
            base = (wid + (i * NEB + k) * NW) * EG
            descs[k].wait()
            _relu_rows(row_b[k])
            pltpu.async_copy(row_b[k], eout.at[pl.ds(base, EG)], wsem[k])
        return carry

    nblk = my_eg // NEB
    lax.fori_loop(0, nblk, edge_blk, 0)
    for k in range(NEB):
        @pl.when(nblk > 0)
        def _drain_tail(_k=k):
            pltpu.make_async_copy(row_b[_k], eout.at[pl.ds(0, EG)],
                                  wsem[_k]).wait()

    rem = my_eg - nblk * NEB
    for t in range(1, NEB):
        @pl.when(rem >= t)
        def _tail(_t=t):
            j = nblk * NEB + _t - 1
            base = (wid + j * NW) * EG
            pltpu.async_copy(tbc_sh.at[eslab.at[pl.ds(j * EG, EG)]],
                             row_b[0], gsem[0]).wait()
            _relu_rows(row_b[0])
            pltpu.sync_copy(row_b[0], eout.at[pl.ds(base, EG)])

    my_ng = (N_NGROUPS - wid + NW - 1) // NW
    slab_b = [nslab, nslab2, nslab]
    nbuf_b = [row_b[0], row_b[1], row_b[0]]
    nsem_b = [gsem[0], gsem[1], gsem[0]]

    def node_fire(i):
        g = wid + i * NW
        pltpu.sync_copy(nidx.at[pl.ds(g * (NA * EG), NA * EG)], slab_b[i])
        ds_ = [pltpu.async_copy(taw_sh.at[slab_b[i].at[pl.ds(0, EG)]],
                                nbuf_b[i], nsem_b[i])]
        for f in range(1, NA):
            ds_.append(pltpu.async_copy(
                taw_sh.at[slab_b[i].at[pl.ds(f * EG, EG)]],
                nbuf_b[i], nsem_b[i], add=True))
        return ds_

    def node_finish(i, ds_):
        g = wid + i * NW
        for d in ds_:
            d.wait()
        _relu_rows(nbuf_b[i])

        @pl.when(g < N_NGROUPS - 1)
        def _full():
            pltpu.sync_copy(nbuf_b[i], nout.at[pl.ds(g * EG, EG)])

        @pl.when(g == N_NGROUPS - 1)
        def _tail16():
            pltpu.sync_copy(nbuf_b[i].at[pl.ds(0, N - (N_NGROUPS - 1) * EG)],
                            nout.at[pl.ds((N_NGROUPS - 1) * EG,
                                          N - (N_NGROUPS - 1) * EG)])

    d0_ = node_fire(0)
    d1_ = node_fire(1)
    node_finish(0, d0_)

    @pl.when(my_ng == 3)
    def _third():
        d2_ = node_fire(2)
        node_finish(1, d1_)
        node_finish(2, d2_)

    @pl.when(my_ng == 2)
    def _no_third():
        node_finish(1, d1_)


def kernel(nodes, edges, receivers, senders, node_graph_idx, edge_graph_idx,
           atom_tables, bond_tables, W_edge, b_edge, W_node, b_node,
           global_table):
    e0 = edges[:, 0].reshape(E // D, D)
    e1 = edges[:, 1].reshape(E // D, D)
    e2 = edges[:, 2].reshape(E // D, D)
    nodes_t = jnp.pad(nodes.T, ((0, 0), (0, NPAD - N))).reshape(
        NA * N_NGROUPS, EG)

    taw, tbc, glat, cidx2d, nidx2d = pl.pallas_call(
        _prep_body,
        out_shape=(
            jax.ShapeDtypeStruct((NA * AV, D), jnp.float32),
            jax.ShapeDtypeStruct((BV * BV * BV, D), jnp.float32),
            jax.ShapeDtypeStruct((B, D), jnp.float32),
            jax.ShapeDtypeStruct((E // D, D), jnp.int32),
            jax.ShapeDtypeStruct((NA * N_NGROUPS, EG), jnp.int32),
        ),
    )(atom_tables, bond_tables, W_edge, b_edge.reshape(1, D),
      W_node, b_node.reshape(1, D), global_table,
      e0, e1, e2, nodes_t)

    cidx_wm = jnp.pad(cidx2d, ((0, EPW * NW - N_EGROUPS), (0, 0)))
    cidx = cidx_wm.reshape(EPW, NW, EG).transpose(1, 0, 2).reshape(
        EPW * NW * EG)
    nidx = nidx2d.reshape(NA, N_NGROUPS, EG).transpose(1, 0, 2).reshape(
        NA * NPAD)

    mesh = plsc.VectorSubcoreMesh(core_axis_name="c", subcore_axis_name="s",
                                  num_cores=NC, num_subcores=NS)
    sc = functools.partial(
        pl.kernel,
        out_type=(
            jax.ShapeDtypeStruct((E, D), jnp.float32),
            jax.ShapeDtypeStruct((N, D), jnp.float32),
        ),
        mesh=mesh,
        scratch_types=(
            [pltpu.VMEM((EPW * EG,), jnp.int32)]
            + [pltpu.VMEM((NA * EG,), jnp.int32)] * 2
            + [pltpu.VMEM((EG, D), jnp.float32)] * 4
            + [pltpu.VMEM_SHARED((BV * BV * BV, D), jnp.float32)]
            + [pltpu.VMEM_SHARED((NA * AV, D), jnp.float32)]
            + [pltpu.SemaphoreType.DMA] * 8
        ),
    )(_sc_body)

    edges_update, nodes_update = sc(tbc, taw, cidx, nidx)

    return (nodes_update, edges_update, receivers, senders, glat,
            node_graph_idx, edge_graph_idx)
